# baseline (speedup 1.0000x reference)
import functools

import jax
import jax.numpy as jnp
from jax import lax
from jax.experimental import pallas as pl
from jax.experimental.pallas import tpu as pltpu
from jax.experimental.pallas import tpu_sc as plsc

_TEMP = 0.5
_K = 256
_N = 96 * 96 * 96
_ROWS = 64
_BLK = 256
_NBLK = _N // _BLK
_NBV = _NBLK // 16
_CAP = 2048
_IMIN = -2147483648


def _key_of(v):
    u = lax.bitcast_convert_type(v, jnp.int32)
    return jnp.where(u >= 0, u, u ^ jnp.int32(0x7FFFFFFF))


def _val_of_key(k):
    u = jnp.where(k >= 0, k, k ^ jnp.int32(0x7FFFFFFF))
    return lax.bitcast_convert_type(u, jnp.float32)


def _bmax_kernel(x_ref, o_ref):
    x = x_ref[...]
    o_ref[...] = jnp.max(x, axis=2).reshape(1, 1, _NBLK)


def _sc_kernel(x_hbm, bm_hbm, o_hbm, blk0, blk1, bmbuf, wl, ck, ci, outv,
               sptr, skey, smax, snw, sem0, sem1):
    lanes = lax.iota(jnp.int32, 16)

    def count_ge(cand, ptr):
        nv = (ptr + 15) // 16

        def cbody(i, acc):
            kv = ck[pl.ds(i * 16, 16)]
            ok = jnp.logical_and(kv >= cand, i * 16 + lanes < ptr)
            return acc + jnp.where(ok, 1, 0).astype(jnp.int32)

        acc = lax.fori_loop(0, nv, cbody, jnp.zeros((16,), jnp.int32))
        return jnp.sum(acc)

    def kth_key(ptr, stop_cnt):
        def body(i, c):
            p, cnt = c

            def live(_):
                cand = p + (jnp.int32(1) << (31 - i))
                cn = count_ge(cand, ptr)
                take = cn >= _K
                return (jnp.where(take, cand, p), jnp.where(take, cn, cnt))

            done = jnp.logical_and(cnt >= _K, cnt <= stop_cnt)
            return lax.cond(done, lambda _: (p, cnt), live, 0)

        p, cnt = lax.fori_loop(0, 32, body,
                               (jnp.int32(_IMIN), jnp.int32(0x7FFFFFFF)))
        return p, cnt

    def count_ge_bm(cand):
        def cbody(i, acc):
            kv = _key_of(bmbuf[pl.ds(i * 16, 16)])
            return acc + jnp.where(kv >= cand, 1, 0).astype(jnp.int32)

        acc = lax.fori_loop(0, _NBV, cbody, jnp.zeros((16,), jnp.int32))
        return jnp.sum(acc)

    def t0_key():
        def body(i, c):
            p, cnt = c

            def live(_):
                cand = p + (jnp.int32(1) << (31 - i))
                cn = count_ge_bm(cand)
                take = cn >= _K
                return (jnp.where(take, cand, p), jnp.where(take, cn, cnt))

            done = jnp.logical_and(cnt >= _K, cnt <= 4 * _K)
            return lax.cond(done, lambda _: c, live, 0)

        p, _ = lax.fori_loop(0, 32, body,
                             (jnp.int32(_IMIN), jnp.int32(0x7FFFFFFF)))
        return p

    def row_max():
        def body(i, acc):
            return jnp.maximum(acc, bmbuf[pl.ds(i * 16, 16)])

        v = lax.fori_loop(0, _NBV, body,
                          jnp.full((16,), -jnp.inf, jnp.float32))
        return jnp.max(v)

    def build_wl(tkey):
        def body(g, wp):
            kv = _key_of(bmbuf[pl.ds(g * 16, 16)])
            msk = kv >= tkey
            plsc.store_compressed(wl.at[pl.ds(wp, 16)], g * 16 + lanes,
                                  mask=msk)
            return wp + jnp.sum(jnp.where(msk, 1, 0).astype(jnp.int32))

        return lax.fori_loop(0, _NBV, body, jnp.int32(0))

    def wl_at(i):
        v = wl[pl.ds((i // 16) * 16, 16)]
        return jnp.max(jnp.where(lanes == i % 16, v, jnp.int32(_IMIN)))

    def reselect():
        ptr = sptr[0]
        tnew, _ = kth_key(ptr, 2 * _K)
        skey[0] = tnew

        nv = (ptr + 15) // 16

        def comp(i, wp):
            kv = ck[pl.ds(i * 16, 16)]
            iv = ci[pl.ds(i * 16, 16)]
            msk = jnp.logical_and(kv >= tnew, i * 16 + lanes < ptr)
            plsc.store_compressed(ck.at[pl.ds(wp, 16)], kv, mask=msk)
            plsc.store_compressed(ci.at[pl.ds(wp, 16)], iv, mask=msk)
            return wp + jnp.sum(jnp.where(msk, 1, 0).astype(jnp.int32))

        sptr[0] = lax.fori_loop(0, nv, comp, jnp.int32(0))

    def process_blk(buf, bid):
        @pl.when(sptr[0] > _CAP - (_BLK + 16))
        def _():
            reselect()

        tkey = skey[0]
        ptr = sptr[0]
        gbase = bid * _BLK
        for j in range(_BLK // 16):
            v = buf[pl.ds(j * 16, 16)]
            kv = _key_of(v)
            msk = kv >= tkey
            plsc.store_compressed(ck.at[pl.ds(ptr, 16)], kv, mask=msk)
            plsc.store_compressed(ci.at[pl.ds(ptr, 16)],
                                  gbase + j * 16 + lanes, mask=msk)
            ptr = ptr + jnp.sum(jnp.where(msk, 1, 0).astype(jnp.int32))
        sptr[0] = ptr

    def do_row(row):
        sptr[0] = jnp.int32(0)

        pltpu.sync_copy(bm_hbm.at[row], bmbuf)
        smax[0] = row_max()
        t0 = t0_key()
        skey[0] = t0
        nw = build_wl(t0)
        snw[0] = nw

        pltpu.async_copy(x_hbm.at[row, pl.ds(wl_at(0) * _BLK, _BLK)],
                         blk0, sem0)

        def bloop(i, _):
            def go(cur, cursem, nxt, nxtsem):
                pltpu.make_async_copy(
                    x_hbm.at[row, pl.ds(0, _BLK)], cur, cursem).wait()

                @pl.when(i + 1 < nw)
                def _():
                    pltpu.async_copy(
                        x_hbm.at[row, pl.ds(wl_at(i + 1) * _BLK, _BLK)],
                        nxt, nxtsem)

                process_blk(cur, wl_at(i))

            @pl.when(i % 2 == 0)
            def _():
                go(blk0, sem0, blk1, sem1)

            @pl.when(i % 2 == 1)
            def _():
                go(blk1, sem1, blk0, sem0)

            return 0

        lax.fori_loop(0, nw, bloop, jnp.int32(0))

        ptr = sptr[0]
        tstar, _ = kth_key(ptr, _K)
        m = smax[0]
        nv = (ptr + 15) // 16

        def wbody(i, accs):
            dgt, ddt, dht, dwt, deq, det, het, wet, ngt, neq = accs
            kv = ck[pl.ds(i * 16, 16)]
            iv = ci[pl.ds(i * 16, 16)]
            occ = i * 16 + lanes < ptr
            gt = jnp.logical_and(kv > tstar, occ)
            eq = jnp.logical_and(kv == tstar, occ)
            ge = jnp.logical_or(gt, eq)
            vv = _val_of_key(kv)
            e = jnp.where(ge, jnp.exp((vv - m) * (1.0 / _TEMP)),
                          jnp.float32(0.0))
            d = (iv // 9216).astype(jnp.float32)
            rem = iv - (iv // 9216) * 9216
            h = (rem // 96).astype(jnp.float32)
            wc = (rem - (rem // 96) * 96).astype(jnp.float32)
            egt = jnp.where(gt, e, 0.0)
            eeq = jnp.where(eq, e, 0.0)
            return (dgt + egt, ddt + egt * d, dht + egt * h, dwt + egt * wc,
                    deq + eeq, det + eeq * d, het + eeq * h, wet + eeq * wc,
                    ngt + jnp.where(gt, 1, 0).astype(jnp.int32),
                    neq + jnp.where(eq, 1, 0).astype(jnp.int32))

        z = jnp.zeros((16,), jnp.float32)
        zi = jnp.zeros((16,), jnp.int32)
        accs = lax.fori_loop(0, nv, wbody,
                             (z, z, z, z, z, z, z, z, zi, zi))
        dgt, ddt, dht, dwt, deq, det, het, wet, ngt, neq = accs
        n_gt = jnp.sum(ngt)
        n_eq = jnp.sum(neq)
        fv = (jnp.full((16,), jnp.int32(_K) - n_gt, jnp.int32)
              .astype(jnp.float32) /
              jnp.full((16,), jnp.maximum(n_eq, 1), jnp.int32)
              .astype(jnp.float32))
        den_v = (jnp.full((16,), jnp.sum(dgt), jnp.float32)
                 + fv * jnp.full((16,), jnp.sum(deq), jnp.float32) + 1e-20)
        num_gt = jnp.where(lanes == 0, jnp.sum(ddt),
                           jnp.where(lanes == 1, jnp.sum(dht),
                                     jnp.where(lanes == 2, jnp.sum(dwt),
                                               0.0)))
        num_eq = jnp.where(lanes == 0, jnp.sum(det),
                           jnp.where(lanes == 1, jnp.sum(het),
                                     jnp.where(lanes == 2, jnp.sum(wet),
                                               0.0)))
        outv[...] = (num_gt + fv * num_eq) / den_v
        pltpu.sync_copy(outv, o_hbm.at[row])

    wid = lax.axis_index("s") * 2 + lax.axis_index("c")

    def rows(r, _):
        do_row(wid * 2 + r)
        return 0

    lax.fori_loop(0, 2, rows, jnp.int32(0))


def kernel(heatmap):
    B, C, D, H, W = heatmap.shape
    x = heatmap.reshape(B * C, _N)

    bm = pl.pallas_call(
        _bmax_kernel,
        grid=(_ROWS,),
        in_specs=[pl.BlockSpec((1, _NBLK, _BLK), lambda r: (r, 0, 0))],
        out_specs=pl.BlockSpec((1, 1, _NBLK), lambda r: (r, 0, 0)),
        out_shape=jax.ShapeDtypeStruct((_ROWS, 1, _NBLK), jnp.float32),
    )(x.reshape(_ROWS, _NBLK, _BLK)).reshape(_ROWS, _NBLK)

    mesh = plsc.VectorSubcoreMesh(core_axis_name="c", subcore_axis_name="s")
    f = functools.partial(
        pl.kernel,
        mesh=mesh,
        out_type=jax.ShapeDtypeStruct((_ROWS, 16), jnp.float32),
        scratch_types=[
            pltpu.VMEM((_BLK,), jnp.float32),
            pltpu.VMEM((_BLK,), jnp.float32),
            pltpu.VMEM((_NBLK,), jnp.float32),
            pltpu.VMEM((_NBLK + 16,), jnp.int32),
            pltpu.VMEM((_CAP,), jnp.int32),
            pltpu.VMEM((_CAP,), jnp.int32),
            pltpu.VMEM((16,), jnp.float32),
            pltpu.SMEM((1,), jnp.int32),
            pltpu.SMEM((1,), jnp.int32),
            pltpu.SMEM((1,), jnp.float32),
            pltpu.SMEM((1,), jnp.int32),
            pltpu.SemaphoreType.DMA,
            pltpu.SemaphoreType.DMA,
        ],
        compiler_params=pltpu.CompilerParams(needs_layout_passes=False),
    )(_sc_kernel)
    out = f(x, bm)
    return out[:, :3].reshape(B, C, 3)


# speedup vs baseline: 5.1787x; 5.1787x over previous
import functools

import jax
import jax.numpy as jnp
from jax import lax
from jax.experimental import pallas as pl
from jax.experimental.pallas import tpu as pltpu
from jax.experimental.pallas import tpu_sc as plsc

_TEMP = 0.5
_K = 256
_N = 96 * 96 * 96
_ROWS = 64
_BLK = 96
_NBLK = _N // _BLK
_NBV = _NBLK // 16
_CAP = 2048
_IMIN = -2147483648


def _key_of(v):
    u = lax.bitcast_convert_type(v, jnp.int32)
    return jnp.where(u >= 0, u, u ^ jnp.int32(0x7FFFFFFF))


def _val_of_key(k):
    u = jnp.where(k >= 0, k, k ^ jnp.int32(0x7FFFFFFF))
    return lax.bitcast_convert_type(u, jnp.float32)


def _bmax_kernel(x_ref, o_ref):
    x = x_ref[...]
    o_ref[...] = jnp.max(x, axis=3).reshape(1, 1, _NBLK)


def _meta_kernel(bm_ref, o_ref):
    keys = _key_of(bm_ref[...].reshape(_ROWS, _NBLK))
    rmax = jnp.max(keys, axis=1, keepdims=True)

    def body(i, p):
        cand = p + (jnp.int32(1) << (31 - i))
        cnt = jnp.sum(jnp.where(keys >= cand, 1, 0).astype(jnp.int32),
                      axis=1, keepdims=True)
        return jnp.where(cnt >= _K, cand, p)

    p = lax.fori_loop(0, 32, body, jnp.full((_ROWS, 1), _IMIN, jnp.int32))
    l = lax.broadcasted_iota(jnp.int32, (_ROWS, 128), 1)
    o_ref[...] = jnp.where(l == 0, p,
                           jnp.where(l == 1, rmax, 0)).reshape(_ROWS, 1, 128)


def _sc_kernel(x_hbm, bm_hbm, mt_hbm, o_hbm,
               blk0, blk1, blk2, blk3, bmbuf, mtbuf, wl, ck, ci, outv,
               sptr, skey, smax, snw, sem0, sem1, sem2, sem3):
    lanes = lax.iota(jnp.int32, 16)

    def count_ge(cand, ptr):
        nv = (ptr + 15) // 16

        def cbody(i, acc):
            kv = ck[pl.ds(i * 16, 16)]
            ok = jnp.logical_and(kv >= cand, i * 16 + lanes < ptr)
            return acc + jnp.where(ok, 1, 0).astype(jnp.int32)

        acc = lax.fori_loop(0, nv, cbody, jnp.zeros((16,), jnp.int32))
        return jnp.sum(acc)

    def kth_key(ptr, stop_cnt):
        def body(i, c):
            p, cnt = c

            def live(_):
                cand = p + (jnp.int32(1) << (31 - i))
                cn = count_ge(cand, ptr)
                take = cn >= _K
                return (jnp.where(take, cand, p), jnp.where(take, cn, cnt))

            done = jnp.logical_and(cnt >= _K, cnt <= stop_cnt)
            return lax.cond(done, lambda _: (p, cnt), live, 0)

        p, cnt = lax.fori_loop(0, 32, body,
                               (jnp.int32(_IMIN), jnp.int32(0x7FFFFFFF)))
        return p, cnt

    def build_wl(tkey):
        def body(g, wp):
            kv = _key_of(bmbuf[pl.ds(g * 16, 16)])
            msk = kv >= tkey
            plsc.store_compressed(wl.at[pl.ds(wp, 16)], g * 16 + lanes,
                                  mask=msk)
            return wp + jnp.sum(jnp.where(msk, 1, 0).astype(jnp.int32))

        return lax.fori_loop(0, _NBV, body, jnp.int32(0))

    def wl_at(i):
        v = wl[pl.ds((i // 16) * 16, 16)]
        return jnp.max(jnp.where(lanes == i % 16, v, jnp.int32(_IMIN)))

    def reselect():
        ptr = sptr[0]
        tnew, _ = kth_key(ptr, 2 * _K)
        skey[0] = tnew

        nv = (ptr + 15) // 16

        def comp(i, wp):
            kv = ck[pl.ds(i * 16, 16)]
            iv = ci[pl.ds(i * 16, 16)]
            msk = jnp.logical_and(kv >= tnew, i * 16 + lanes < ptr)
            plsc.store_compressed(ck.at[pl.ds(wp, 16)], kv, mask=msk)
            plsc.store_compressed(ci.at[pl.ds(wp, 16)], iv, mask=msk)
            return wp + jnp.sum(jnp.where(msk, 1, 0).astype(jnp.int32))

        sptr[0] = lax.fori_loop(0, nv, comp, jnp.int32(0))

    def process_blk(buf, bid):
        @pl.when(sptr[0] > _CAP - (_BLK + 16))
        def _():
            reselect()

        tkey = skey[0]
        ptr = sptr[0]
        gbase = bid * _BLK
        for j in range(_BLK // 16):
            v = buf[pl.ds(j * 16, 16)]
            kv = _key_of(v)
            msk = kv >= tkey
            plsc.store_compressed(ck.at[pl.ds(ptr, 16)], kv, mask=msk)
            plsc.store_compressed(ci.at[pl.ds(ptr, 16)],
                                  gbase + j * 16 + lanes, mask=msk)
            ptr = ptr + jnp.sum(jnp.where(msk, 1, 0).astype(jnp.int32))
        sptr[0] = ptr

    def do_row(row):
        sptr[0] = jnp.int32(0)

        pltpu.sync_copy(bm_hbm.at[row, 0], bmbuf)
        pltpu.sync_copy(mt_hbm.at[row, 0], mtbuf)
        mv = mtbuf[pl.ds(0, 16)]
        t0 = jnp.max(jnp.where(lanes == 0, mv, jnp.int32(_IMIN)))
        smax[0] = jnp.max(jnp.where(lanes == 1, _val_of_key(mv),
                                    jnp.float32(-jnp.inf)))
        skey[0] = t0
        nw = build_wl(t0)
        snw[0] = nw

        for j, (b, s) in enumerate(((blk0, sem0), (blk1, sem1),
                                    (blk2, sem2), (blk3, sem3))):
            @pl.when(j < nw)
            def _(b=b, s=s, j=j):
                pltpu.async_copy(x_hbm.at[row, wl_at(j)], b, s)

        def bloop(i, _):
            def go(buf, sem):
                pltpu.make_async_copy(x_hbm.at[row, 0], buf, sem).wait()
                process_blk(buf, wl_at(i))

                @pl.when(i + 4 < nw)
                def _():
                    pltpu.async_copy(x_hbm.at[row, wl_at(i + 4)], buf, sem)

            for k, (b, s) in enumerate(((blk0, sem0), (blk1, sem1),
                                        (blk2, sem2), (blk3, sem3))):
                @pl.when(i % 4 == k)
                def _(b=b, s=s):
                    go(b, s)

            return 0

        lax.fori_loop(0, nw, bloop, jnp.int32(0))

        ptr = sptr[0]
        tstar, _ = kth_key(ptr, _K)
        m = smax[0]
        nv = (ptr + 15) // 16

        def wbody(i, accs):
            dgt, ddt, dht, dwt, deq, det, het, wet, ngt, neq = accs
            kv = ck[pl.ds(i * 16, 16)]
            iv = ci[pl.ds(i * 16, 16)]
            occ = i * 16 + lanes < ptr
            gt = jnp.logical_and(kv > tstar, occ)
            eq = jnp.logical_and(kv == tstar, occ)
            ge = jnp.logical_or(gt, eq)
            vv = _val_of_key(kv)
            e = jnp.where(ge, jnp.exp((vv - m) * (1.0 / _TEMP)),
                          jnp.float32(0.0))
            d = (iv // 9216).astype(jnp.float32)
            rem = iv - (iv // 9216) * 9216
            h = (rem // 96).astype(jnp.float32)
            wc = (rem - (rem // 96) * 96).astype(jnp.float32)
            egt = jnp.where(gt, e, 0.0)
            eeq = jnp.where(eq, e, 0.0)
            return (dgt + egt, ddt + egt * d, dht + egt * h, dwt + egt * wc,
                    deq + eeq, det + eeq * d, het + eeq * h, wet + eeq * wc,
                    ngt + jnp.where(gt, 1, 0).astype(jnp.int32),
                    neq + jnp.where(eq, 1, 0).astype(jnp.int32))

        z = jnp.zeros((16,), jnp.float32)
        zi = jnp.zeros((16,), jnp.int32)
        accs = lax.fori_loop(0, nv, wbody,
                             (z, z, z, z, z, z, z, z, zi, zi))
        dgt, ddt, dht, dwt, deq, det, het, wet, ngt, neq = accs
        n_gt = jnp.sum(ngt)
        n_eq = jnp.sum(neq)
        fv = (jnp.full((16,), jnp.int32(_K) - n_gt, jnp.int32)
              .astype(jnp.float32) /
              jnp.full((16,), jnp.maximum(n_eq, 1), jnp.int32)
              .astype(jnp.float32))
        den_v = (jnp.full((16,), jnp.sum(dgt), jnp.float32)
                 + fv * jnp.full((16,), jnp.sum(deq), jnp.float32) + 1e-20)
        num_gt = jnp.where(lanes == 0, jnp.sum(ddt),
                           jnp.where(lanes == 1, jnp.sum(dht),
                                     jnp.where(lanes == 2, jnp.sum(dwt),
                                               0.0)))
        num_eq = jnp.where(lanes == 0, jnp.sum(det),
                           jnp.where(lanes == 1, jnp.sum(het),
                                     jnp.where(lanes == 2, jnp.sum(wet),
                                               0.0)))
        outv[...] = (num_gt + fv * num_eq) / den_v
        pltpu.sync_copy(outv, o_hbm.at[row])

    wid = lax.axis_index("s") * 2 + lax.axis_index("c")

    def rows(r, _):
        do_row(wid * 2 + r)
        return 0

    lax.fori_loop(0, 2, rows, jnp.int32(0))


def kernel(heatmap):
    B, C, D, H, W = heatmap.shape
    x4 = heatmap.reshape(B * C, D, H, W)

    bm = pl.pallas_call(
        _bmax_kernel,
        grid=(_ROWS,),
        in_specs=[pl.BlockSpec((1, D, H, W), lambda r: (r, 0, 0, 0))],
        out_specs=pl.BlockSpec((1, 1, _NBLK), lambda r: (r, 0, 0)),
        out_shape=jax.ShapeDtypeStruct((_ROWS, 1, _NBLK), jnp.float32),
    )(x4)

    mt = pl.pallas_call(
        _meta_kernel,
        grid=(1,),
        in_specs=[pl.BlockSpec((_ROWS, 1, _NBLK), lambda r: (0, 0, 0))],
        out_specs=pl.BlockSpec((_ROWS, 1, 128), lambda r: (0, 0, 0)),
        out_shape=jax.ShapeDtypeStruct((_ROWS, 1, 128), jnp.int32),
    )(bm)

    x42 = x4.reshape(_ROWS, _NBLK, _BLK)
    mesh = plsc.VectorSubcoreMesh(core_axis_name="c", subcore_axis_name="s")
    f = functools.partial(
        pl.kernel,
        mesh=mesh,
        out_type=jax.ShapeDtypeStruct((_ROWS, 16), jnp.float32),
        scratch_types=[
            pltpu.VMEM((_BLK,), jnp.float32),
            pltpu.VMEM((_BLK,), jnp.float32),
            pltpu.VMEM((_BLK,), jnp.float32),
            pltpu.VMEM((_BLK,), jnp.float32),
            pltpu.VMEM((_NBLK,), jnp.float32),
            pltpu.VMEM((128,), jnp.int32),
            pltpu.VMEM((_NBLK + 16,), jnp.int32),
            pltpu.VMEM((_CAP,), jnp.int32),
            pltpu.VMEM((_CAP,), jnp.int32),
            pltpu.VMEM((16,), jnp.float32),
            pltpu.SMEM((1,), jnp.int32),
            pltpu.SMEM((1,), jnp.int32),
            pltpu.SMEM((1,), jnp.float32),
            pltpu.SMEM((1,), jnp.int32),
            pltpu.SemaphoreType.DMA,
            pltpu.SemaphoreType.DMA,
            pltpu.SemaphoreType.DMA,
            pltpu.SemaphoreType.DMA,
        ],
        compiler_params=pltpu.CompilerParams(needs_layout_passes=False),
    )(_sc_kernel)
    out = f(x42, bm, mt)
    return out[:, :3].reshape(B, C, 3)
